# baseline (speedup 1.0000x reference)
import jax
import jax.numpy as jnp
from jax.experimental import pallas as pl

HIDDEN = 2048
NUM_EXPERTS = 64
TOKENS = 16384
NSTREAM = 8
BT = 256
PART = TOKENS // NSTREAM


def _top2_softmax(logits, wout_ref, iout_ref):
    idx = jax.lax.broadcasted_iota(jnp.int32, logits.shape, 1)
    m0 = jnp.max(logits, axis=-1, keepdims=True)
    i0 = jnp.min(jnp.where(logits == m0, idx, NUM_EXPERTS), axis=-1,
                 keepdims=True)
    masked = jnp.where(idx == i0, -jnp.inf, logits)
    m1 = jnp.max(masked, axis=-1, keepdims=True)
    i1 = jnp.min(jnp.where(masked == m1, idx, NUM_EXPERTS), axis=-1,
                 keepdims=True)
    e1 = jnp.exp(m1 - m0)
    denom = 1.0 + e1
    wout_ref[...] = jnp.concatenate([1.0 / denom, e1 / denom], axis=-1)
    iout_ref[...] = jnp.concatenate([i0, i1], axis=-1)


def _router_block(*refs):
    x_refs = refs[:NSTREAM]
    w = refs[NSTREAM][...]
    out_refs = refs[NSTREAM + 1:]
    dn = (((1,), (1,)), ((), ()))
    for s in range(NSTREAM):
        logits = jax.lax.dot_general(x_refs[s][0], w, dimension_numbers=dn,
                                     preferred_element_type=jnp.float32)
        _top2_softmax(logits, out_refs[2 * s], out_refs[2 * s + 1])


@jax.jit
def kernel(x, weight):
    grid = (PART // BT,)
    x4 = x.reshape(NSTREAM, PART, HIDDEN)

    def make_xspec(s):
        return pl.BlockSpec((1, BT, HIDDEN), lambda i, s=s: (s, i, 0))

    out_specs, out_shape = [], []
    for _ in range(NSTREAM):
        out_specs += [pl.BlockSpec((BT, 2), lambda i: (i, 0)),
                      pl.BlockSpec((BT, 2), lambda i: (i, 0))]
        out_shape += [jax.ShapeDtypeStruct((PART, 2), jnp.float32),
                      jax.ShapeDtypeStruct((PART, 2), jnp.int32)]

    outs = pl.pallas_call(
        _router_block,
        grid=grid,
        in_specs=[make_xspec(s) for s in range(NSTREAM)]
        + [pl.BlockSpec((NUM_EXPERTS, HIDDEN), lambda i: (0, 0))],
        out_specs=out_specs,
        out_shape=out_shape,
    )(*([x4] * NSTREAM), weight)
    return (jnp.concatenate(outs[0::2], axis=0),
            jnp.concatenate(outs[1::2], axis=0))


# speedup vs baseline: 1.2280x; 1.2280x over previous
import jax
import jax.numpy as jnp
from jax.experimental import pallas as pl

HIDDEN = 2048
NUM_EXPERTS = 64
TOKENS = 16384
NSTREAM = 4
BT = 512
PART = TOKENS // NSTREAM


def _top2_softmax(logits):
    idx = jax.lax.broadcasted_iota(jnp.int32, logits.shape, 1)
    m0 = jnp.max(logits, axis=-1, keepdims=True)
    i0 = jnp.min(jnp.where(logits == m0, idx, NUM_EXPERTS), axis=-1,
                 keepdims=True)
    masked = jnp.where(idx == i0, -jnp.inf, logits)
    m1 = jnp.max(masked, axis=-1, keepdims=True)
    i1 = jnp.min(jnp.where(masked == m1, idx, NUM_EXPERTS), axis=-1,
                 keepdims=True)
    e1 = jnp.exp(m1 - m0)
    denom = 1.0 + e1
    return (jnp.concatenate([1.0 / denom, e1 / denom], axis=-1),
            jnp.concatenate([i0, i1], axis=-1))


def _router_block(*refs):
    x_refs = refs[:NSTREAM]
    w = refs[NSTREAM][...]
    wout_ref, iout_ref = refs[NSTREAM + 1], refs[NSTREAM + 2]
    dn = (((1,), (1,)), ((), ()))
    for s in range(NSTREAM):
        logits = jax.lax.dot_general(x_refs[s][0], w, dimension_numbers=dn,
                                     preferred_element_type=jnp.float32)
        wv, iv = _top2_softmax(logits)
        wout_ref[s] = wv
        iout_ref[s] = iv


@jax.jit
def kernel(x, weight):
    grid = (PART // BT,)
    x4 = x.reshape(NSTREAM, PART, HIDDEN)

    def make_xspec(s):
        return pl.BlockSpec((1, BT, HIDDEN), lambda i, s=s: (s, i, 0))

    wout, iout = pl.pallas_call(
        _router_block,
        grid=grid,
        in_specs=[make_xspec(s) for s in range(NSTREAM)]
        + [pl.BlockSpec((NUM_EXPERTS, HIDDEN), lambda i: (0, 0))],
        out_specs=[
            pl.BlockSpec((NSTREAM, BT, 2), lambda i: (0, i, 0)),
            pl.BlockSpec((NSTREAM, BT, 2), lambda i: (0, i, 0)),
        ],
        out_shape=[
            jax.ShapeDtypeStruct((NSTREAM, PART, 2), jnp.float32),
            jax.ShapeDtypeStruct((NSTREAM, PART, 2), jnp.int32),
        ],
    )(*([x4] * NSTREAM), weight)
    return (wout.reshape(TOKENS, 2), iout.reshape(TOKENS, 2))


# speedup vs baseline: 1.2430x; 1.0122x over previous
import jax
import jax.numpy as jnp
from jax.experimental import pallas as pl

HIDDEN = 2048
NUM_EXPERTS = 64
TOKENS = 16384
NSTREAM = 4
BT = 512
PART = TOKENS // NSTREAM


def _top2_softmax(logits, wout_ref, iout_ref):
    idx = jax.lax.broadcasted_iota(jnp.int32, logits.shape, 1)
    m0 = jnp.max(logits, axis=-1, keepdims=True)
    i0 = jnp.min(jnp.where(logits == m0, idx, NUM_EXPERTS), axis=-1,
                 keepdims=True)
    masked = jnp.where(idx == i0, -jnp.inf, logits)
    m1 = jnp.max(masked, axis=-1, keepdims=True)
    i1 = jnp.min(jnp.where(masked == m1, idx, NUM_EXPERTS), axis=-1,
                 keepdims=True)
    e1 = jnp.exp(m1 - m0)
    denom = 1.0 + e1
    wout_ref[...] = jnp.concatenate([1.0 / denom, e1 / denom], axis=-1)
    iout_ref[...] = jnp.concatenate([i0, i1], axis=-1)


def _router_block(*refs):
    x_refs = refs[:NSTREAM]
    w = refs[NSTREAM][...]
    out_refs = refs[NSTREAM + 1:]
    dn = (((1,), (1,)), ((), ()))
    for s in range(NSTREAM):
        logits = jax.lax.dot_general(x_refs[s][0], w, dimension_numbers=dn,
                                     preferred_element_type=jnp.float32)
        _top2_softmax(logits, out_refs[2 * s], out_refs[2 * s + 1])


@jax.jit
def kernel(x, weight):
    grid = (PART // BT,)
    x4 = x.reshape(NSTREAM, PART, HIDDEN)

    def make_xspec(s):
        return pl.BlockSpec((1, BT, HIDDEN), lambda i, s=s: (s, i, 0))

    out_specs, out_shape = [], []
    for _ in range(NSTREAM):
        out_specs += [pl.BlockSpec((BT, 2), lambda i: (i, 0)),
                      pl.BlockSpec((BT, 2), lambda i: (i, 0))]
        out_shape += [jax.ShapeDtypeStruct((PART, 2), jnp.float32),
                      jax.ShapeDtypeStruct((PART, 2), jnp.int32)]

    outs = pl.pallas_call(
        _router_block,
        grid=grid,
        in_specs=[make_xspec(s) for s in range(NSTREAM)]
        + [pl.BlockSpec((NUM_EXPERTS, HIDDEN), lambda i: (0, 0))],
        out_specs=out_specs,
        out_shape=out_shape,
    )(*([x4] * NSTREAM), weight)
    return (jnp.concatenate(outs[0::2], axis=0),
            jnp.concatenate(outs[1::2], axis=0))
